# baseline (speedup 1.0000x reference)
import functools

import jax
import jax.numpy as jnp
import numpy as np
from jax import lax
from jax.experimental import pallas as pl
from jax.experimental.pallas import tpu as pltpu
from jax.experimental.pallas import tpu_sc as plsc

_BATCH = 16384
_DIM = 32
_ROWS = 1000000
_PACK = 128 // _DIM
_WROWS = _ROWS // _PACK
_NC = 2
_NS = 16
_NW = _NC * _NS
_BPW = _BATCH // _NW
_GCH = 128
_NCH = _BPW // _GCH


def _sc_scores_body(uidx_hbm, iidx_hbm, utab_hbm, itab_hbm, btab_hbm,
                    out_hbm,
                    uix_v, iix_v, qu_v, qi_v,
                    ubuf0, ubuf1, ibuf0, ibuf1,
                    bias_v, score_v, sem0, sem1):
    wid = lax.axis_index("s") * _NC + lax.axis_index("c")
    base = wid * _BPW

    pltpu.sync_copy(uidx_hbm.at[pl.ds(base, _BPW)], uix_v)
    pltpu.sync_copy(iidx_hbm.at[pl.ds(base, _BPW)], iix_v)

    def shift_body(g, carry):
        sl = pl.ds(g * 16, 16)
        qu_v[sl] = lax.shift_right_logical(uix_v[sl], 2)
        qi_v[sl] = lax.shift_right_logical(iix_v[sl], 2)
        return carry

    lax.fori_loop(0, _BPW // 16, shift_body, 0)

    ubufs = (ubuf0, ubuf1)
    ibufs = (ibuf0, ibuf1)
    sems = (sem0, sem1)

    def chunk_copies(j):
        sl = pl.ds(j * _GCH, _GCH)
        sem = sems[j % 2]
        return (
            pltpu.make_async_copy(utab_hbm.at[qu_v.at[sl]], ubufs[j % 2], sem),
            pltpu.make_async_copy(itab_hbm.at[qi_v.at[sl]], ibufs[j % 2], sem),
            pltpu.make_async_copy(btab_hbm.at[iix_v.at[sl]], bias_v.at[sl], sem),
        )

    all_copies = [chunk_copies(j) for j in range(_NCH)]
    for c in all_copies[0]:
        c.start()

    for j in range(_NCH):
        if j + 1 < _NCH:
            for c in all_copies[j + 1]:
                c.start()
        for c in all_copies[j]:
            c.wait()

        ub = ubufs[j % 2]
        ib = ibufs[j % 2]

        def g_body(g, carry):
            e0 = j * _GCH + g * 16
            esl = pl.ds(e0, 16)
            rows16 = g * 16 + lax.iota(jnp.int32, 16)
            ou = (uix_v[esl] & 3) * _DIM
            oi = (iix_v[esl] & 3) * _DIM
            acc = bias_v[esl]
            for d in range(_DIM):
                uv = plsc.load_gather(ub, [rows16, ou + d])
                iv = plsc.load_gather(ib, [rows16, oi + d])
                acc = acc + uv * iv
            score_v[esl] = acc
            return carry

        lax.fori_loop(0, _GCH // 16, g_body, 0)

    pltpu.sync_copy(score_v, out_hbm.at[pl.ds(base, _BPW)])


_sc_scores = functools.partial(
    pl.kernel,
    out_type=jax.ShapeDtypeStruct((_BATCH,), jnp.float32),
    mesh=plsc.VectorSubcoreMesh(core_axis_name="c", subcore_axis_name="s"),
    compiler_params=pltpu.CompilerParams(needs_layout_passes=False),
    scratch_types=[
        pltpu.VMEM((_BPW,), jnp.int32),
        pltpu.VMEM((_BPW,), jnp.int32),
        pltpu.VMEM((_BPW,), jnp.int32),
        pltpu.VMEM((_BPW,), jnp.int32),
        pltpu.VMEM((_GCH, 128), jnp.float32),
        pltpu.VMEM((_GCH, 128), jnp.float32),
        pltpu.VMEM((_GCH, 128), jnp.float32),
        pltpu.VMEM((_GCH, 128), jnp.float32),
        pltpu.VMEM((_BPW,), jnp.float32),
        pltpu.VMEM((_BPW,), jnp.float32),
        pltpu.SemaphoreType.DMA,
        pltpu.SemaphoreType.DMA,
    ],
)(_sc_scores_body)


def _loss_body(s_ref, y_ref, o_ref):
    s = s_ref[...]
    y = y_ref[...]
    per = jnp.maximum(s, 0.0) - s * y + jnp.log1p(jnp.exp(-jnp.abs(s)))
    o_ref[...] = jnp.sum(per).reshape(1, 1) / np.float32(_BATCH)


def kernel(input_user, input_item, pred_data_label,
           D_user_embeddings, D_item_embeddings, D_item_bias):
    utab = D_user_embeddings.reshape(_WROWS, 128)
    itab = D_item_embeddings.reshape(_WROWS, 128)
    scores = _sc_scores(input_user, input_item, utab, itab, D_item_bias)
    loss = pl.pallas_call(
        _loss_body,
        out_shape=jax.ShapeDtypeStruct((1, 1), jnp.float32),
    )(scores.reshape(128, 128),
      pred_data_label.astype(jnp.float32).reshape(128, 128))
    return loss[0, 0]


# speedup vs baseline: 3.7378x; 3.7378x over previous
import functools

import jax
import jax.numpy as jnp
import numpy as np
from jax import lax
from jax.experimental import pallas as pl
from jax.experimental.pallas import tpu as pltpu
from jax.experimental.pallas import tpu_sc as plsc

_BATCH = 16384
_DIM = 32
_NC = 2
_NS = 16
_NW = _NC * _NS
_BPW = _BATCH // _NW
_GRP = 16
_NGRP = _BPW // _GRP
_NQ = _DIM // 8
_NRND = _NGRP * _NQ
_GCH = 128


def _sc_scores_body(uidx_hbm, iidx_hbm, utab_hbm, itab_hbm, btab_hbm,
                    out_hbm,
                    uix_v, iix_v, ubuf, ibuf, bias_v, score_v,
                    sem0, sem1, semb):
    wid = lax.axis_index("s") * _NC + lax.axis_index("c")
    base = wid * _BPW

    pltpu.sync_copy(uidx_hbm.at[pl.ds(base, _BPW)], uix_v)
    pltpu.sync_copy(iidx_hbm.at[pl.ds(base, _BPW)], iix_v)

    def bias_copies():
        return [
            pltpu.make_async_copy(
                btab_hbm.at[iix_v.at[pl.ds(j * _GCH, _GCH)]],
                bias_v.at[pl.ds(j * _GCH, _GCH)], semb)
            for j in range(_BPW // _GCH)
        ]

    for c in bias_copies():
        c.start()

    sems = (sem0, sem1)
    bufs = ((ubuf, uix_v, utab_hbm), (ibuf, iix_v, itab_hbm))

    def start_round(t, p):
        g = lax.shift_right_logical(t, 2)
        q8 = pl.multiple_of((t & 3) * 8, 8)
        for buf, ix_v, tab in bufs:
            c0 = (ix_v[pl.ds(g * _GRP, _GRP)] & jnp.int32(-128))
            for l in range(_GRP):
                pltpu.make_async_copy(
                    tab.at[pl.ds(q8, 8), pl.ds(pl.multiple_of(c0[l], 128), 128)],
                    buf.at[p, l], sems[p]).start()

    def wait_round(p):
        for buf, _, tab in bufs:
            for l in range(_GRP):
                pltpu.make_async_copy(
                    tab.at[pl.ds(0, 8), pl.ds(0, 128)],
                    buf.at[p, l], sems[p]).wait()

    def compute_round(t, p):
        g = lax.shift_right_logical(t, 2)
        sl = pl.ds(g * _GRP, _GRP)
        lu = uix_v[sl] & 127
        li = iix_v[sl] & 127
        pv = jnp.full((_GRP,), p, jnp.int32)
        elane = lax.iota(jnp.int32, _GRP)
        acc = score_v[sl]
        for s in range(8):
            sv = jnp.full((_GRP,), s, jnp.int32)
            acc = acc + (plsc.load_gather(ubuf, [pv, elane, sv, lu]) *
                         plsc.load_gather(ibuf, [pv, elane, sv, li]))
        score_v[sl] = acc

    start_round(jnp.int32(0), 0)

    for c in bias_copies():
        c.wait()

    def init_body(g, carry):
        sl = pl.ds(g * _GRP, _GRP)
        score_v[sl] = bias_v[sl]
        return carry

    lax.fori_loop(0, _NGRP, init_body, 0)

    def two_rounds(k, carry):
        start_round(2 * k + 1, 1)
        wait_round(0)
        compute_round(2 * k, 0)

        @pl.when(k < _NRND // 2 - 1)
        def _():
            start_round(2 * k + 2, 0)

        wait_round(1)
        compute_round(2 * k + 1, 1)
        return carry

    lax.fori_loop(0, _NRND // 2, two_rounds, 0)
    pltpu.sync_copy(score_v, out_hbm.at[pl.ds(base, _BPW)])


_sc_scores = functools.partial(
    pl.kernel,
    out_type=jax.ShapeDtypeStruct((_BATCH,), jnp.float32),
    mesh=plsc.VectorSubcoreMesh(core_axis_name="c", subcore_axis_name="s"),
    compiler_params=pltpu.CompilerParams(needs_layout_passes=False),
    scratch_types=[
        pltpu.VMEM((_BPW,), jnp.int32),
        pltpu.VMEM((_BPW,), jnp.int32),
        pltpu.VMEM((2, _GRP, 8, 128), jnp.float32),
        pltpu.VMEM((2, _GRP, 8, 128), jnp.float32),
        pltpu.VMEM((_BPW,), jnp.float32),
        pltpu.VMEM((_BPW,), jnp.float32),
        pltpu.SemaphoreType.DMA,
        pltpu.SemaphoreType.DMA,
        pltpu.SemaphoreType.DMA,
    ],
)(_sc_scores_body)


def _loss_body(s_ref, y_ref, o_ref):
    s = s_ref[...]
    y = y_ref[...]
    per = jnp.maximum(s, 0.0) - s * y + jnp.log1p(jnp.exp(-jnp.abs(s)))
    o_ref[...] = jnp.sum(per).reshape(1, 1) / np.float32(_BATCH)


def kernel(input_user, input_item, pred_data_label,
           D_user_embeddings, D_item_embeddings, D_item_bias):
    scores = _sc_scores(input_user, input_item,
                        D_user_embeddings.T, D_item_embeddings.T, D_item_bias)
    loss = pl.pallas_call(
        _loss_body,
        out_shape=jax.ShapeDtypeStruct((1, 1), jnp.float32),
    )(scores.reshape(128, 128),
      pred_data_label.astype(jnp.float32).reshape(128, 128))
    return loss[0, 0]


# speedup vs baseline: 4.0774x; 1.0908x over previous
import functools

import jax
import jax.numpy as jnp
import numpy as np
from jax import lax
from jax.experimental import pallas as pl
from jax.experimental.pallas import tpu as pltpu
from jax.experimental.pallas import tpu_sc as plsc

_BATCH = 16384
_DIM = 32
_NC = 2
_NS = 16
_NW = _NC * _NS
_BPW = _BATCH // _NW
_GRP = 16
_NGRP = _BPW // _GRP
_NQ = _DIM // 8
_NRND = _NGRP * _NQ
_GCH = 128


def _sc_scores_body(uidx_hbm, iidx_hbm, utab_hbm, itab_hbm, btab_hbm,
                    out_hbm,
                    uix_v, iix_v, ubuf, ibuf, bias_v, score_v,
                    sem0, sem1, sem2, semb):
    wid = lax.axis_index("s") * _NC + lax.axis_index("c")
    base = wid * _BPW

    pltpu.sync_copy(uidx_hbm.at[pl.ds(base, _BPW)], uix_v)
    pltpu.sync_copy(iidx_hbm.at[pl.ds(base, _BPW)], iix_v)

    def bias_copies():
        return [
            pltpu.make_async_copy(
                btab_hbm.at[iix_v.at[pl.ds(j * _GCH, _GCH)]],
                bias_v.at[pl.ds(j * _GCH, _GCH)], semb)
            for j in range(_BPW // _GCH)
        ]

    for c in bias_copies():
        c.start()

    sems = (sem0, sem1, sem2)
    bufs = ((ubuf, uix_v, utab_hbm), (ibuf, iix_v, itab_hbm))

    def start_round(t, p):
        g = lax.shift_right_logical(t, 2)
        q8 = pl.multiple_of((t & 3) * 8, 8)
        for buf, ix_v, tab in bufs:
            c0 = (ix_v[pl.ds(g * _GRP, _GRP)] & jnp.int32(-128))
            for l in range(_GRP):
                pltpu.make_async_copy(
                    tab.at[pl.ds(q8, 8), pl.ds(pl.multiple_of(c0[l], 128), 128)],
                    buf.at[p, l], sems[p]).start()

    def wait_round(p):
        for buf, _, tab in bufs:
            for l in range(_GRP):
                pltpu.make_async_copy(
                    tab.at[pl.ds(0, 8), pl.ds(0, 128)],
                    buf.at[p, l], sems[p]).wait()

    def compute_round(t, p):
        g = lax.shift_right_logical(t, 2)
        sl = pl.ds(g * _GRP, _GRP)
        lu = uix_v[sl] & 127
        li = iix_v[sl] & 127
        pv = jnp.full((_GRP,), p, jnp.int32)
        elane = lax.iota(jnp.int32, _GRP)
        acc = score_v[sl]
        for s in range(8):
            sv = jnp.full((_GRP,), s, jnp.int32)
            acc = acc + (plsc.load_gather(ubuf, [pv, elane, sv, lu]) *
                         plsc.load_gather(ibuf, [pv, elane, sv, li]))
        score_v[sl] = acc

    start_round(jnp.int32(0), 0)
    start_round(jnp.int32(1), 1)

    for c in bias_copies():
        c.wait()

    def init_body(g, carry):
        sl = pl.ds(g * _GRP, _GRP)
        score_v[sl] = bias_v[sl]
        return carry

    lax.fori_loop(0, _NGRP, init_body, 0)

    def three_rounds(k, carry):
        t0 = 3 * k
        start_round(t0 + 2, 2)
        wait_round(0)
        compute_round(t0, 0)

        @pl.when(t0 + 3 < _NRND)
        def _():
            start_round(t0 + 3, 0)

        wait_round(1)
        compute_round(t0 + 1, 1)

        @pl.when(t0 + 4 < _NRND)
        def _():
            start_round(t0 + 4, 1)

        wait_round(2)
        compute_round(t0 + 2, 2)
        return carry

    lax.fori_loop(0, _NRND // 3, three_rounds, 0)
    wait_round(0)
    compute_round(jnp.int32(_NRND - 2), 0)
    wait_round(1)
    compute_round(jnp.int32(_NRND - 1), 1)
    pltpu.sync_copy(score_v, out_hbm.at[pl.ds(base, _BPW)])


_sc_scores = functools.partial(
    pl.kernel,
    out_type=jax.ShapeDtypeStruct((_BATCH,), jnp.float32),
    mesh=plsc.VectorSubcoreMesh(core_axis_name="c", subcore_axis_name="s"),
    compiler_params=pltpu.CompilerParams(needs_layout_passes=False),
    scratch_types=[
        pltpu.VMEM((_BPW,), jnp.int32),
        pltpu.VMEM((_BPW,), jnp.int32),
        pltpu.VMEM((3, _GRP, 8, 128), jnp.float32),
        pltpu.VMEM((3, _GRP, 8, 128), jnp.float32),
        pltpu.VMEM((_BPW,), jnp.float32),
        pltpu.VMEM((_BPW,), jnp.float32),
        pltpu.SemaphoreType.DMA,
        pltpu.SemaphoreType.DMA,
        pltpu.SemaphoreType.DMA,
        pltpu.SemaphoreType.DMA,
    ],
)(_sc_scores_body)


def _loss_body(s_ref, y_ref, o_ref):
    s = s_ref[...]
    y = y_ref[...]
    per = jnp.maximum(s, 0.0) - s * y + jnp.log1p(jnp.exp(-jnp.abs(s)))
    o_ref[...] = jnp.sum(per).reshape(1, 1) / np.float32(_BATCH)


def kernel(input_user, input_item, pred_data_label,
           D_user_embeddings, D_item_embeddings, D_item_bias):
    scores = _sc_scores(input_user, input_item,
                        D_user_embeddings.T, D_item_embeddings.T, D_item_bias)
    loss = pl.pallas_call(
        _loss_body,
        out_shape=jax.ShapeDtypeStruct((1, 1), jnp.float32),
    )(scores.reshape(128, 128),
      pred_data_label.astype(jnp.float32).reshape(128, 128))
    return loss[0, 0]
